# baseline (speedup 1.0000x reference)
import jax
import jax.numpy as jnp
from jax.experimental import pallas as pl
from jax.experimental.pallas import tpu as pltpu

BLOCK_C = 2048


def _body(x_ref, w_ref, b_ref, o_ref):
    acc = jax.lax.dot_general(
        x_ref[...], w_ref[...],
        dimension_numbers=(((1,), (1,)), ((), ())),
        preferred_element_type=jnp.float32,
    )
    o_ref[...] = acc + b_ref[...]


def kernel(x, weight, bias):
    n_classes = weight.shape[0]
    grid = (pl.cdiv(n_classes, BLOCK_C),)
    bias2d = bias.reshape(1, n_classes)
    out = pl.pallas_call(
        _body,
        grid=grid,
        in_specs=[
            pl.BlockSpec((x.shape[0], x.shape[1]), lambda i: (0, 0)),
            pl.BlockSpec((BLOCK_C, weight.shape[1]), lambda i: (i, 0)),
            pl.BlockSpec((1, BLOCK_C), lambda i: (0, i)),
        ],
        out_specs=pl.BlockSpec((x.shape[0], BLOCK_C), lambda i: (0, i)),
        out_shape=jax.ShapeDtypeStruct((x.shape[0], n_classes), jnp.float32),
        compiler_params=pltpu.CompilerParams(
            dimension_semantics=("arbitrary",),
        ),
    )(x, weight, bias2d)
    return out


# speedup vs baseline: 1.8243x; 1.8243x over previous
import jax
import jax.numpy as jnp
from jax.experimental import pallas as pl
from jax.experimental.pallas import tpu as pltpu

BLOCK_C = 8192


def _body(x_ref, w_ref, b_ref, o_ref):
    acc = jax.lax.dot_general(
        x_ref[...], w_ref[...],
        dimension_numbers=(((1,), (1,)), ((), ())),
        preferred_element_type=jnp.float32,
    )
    o_ref[...] = acc + b_ref[...]


def kernel(x, weight, bias):
    n_classes = weight.shape[0]
    grid = (pl.cdiv(n_classes, BLOCK_C),)
    bias2d = bias.reshape(1, n_classes)
    out = pl.pallas_call(
        _body,
        grid=grid,
        in_specs=[
            pl.BlockSpec((x.shape[0], x.shape[1]), lambda i: (0, 0)),
            pl.BlockSpec((BLOCK_C, weight.shape[1]), lambda i: (i, 0)),
            pl.BlockSpec((1, BLOCK_C), lambda i: (0, i)),
        ],
        out_specs=pl.BlockSpec((x.shape[0], BLOCK_C), lambda i: (0, i)),
        out_shape=jax.ShapeDtypeStruct((x.shape[0], n_classes), jnp.float32),
        compiler_params=pltpu.CompilerParams(
            dimension_semantics=("parallel",),
        ),
    )(x, weight, bias2d)
    return out


# speedup vs baseline: 1.9927x; 1.0923x over previous
import jax
import jax.numpy as jnp
from jax.experimental import pallas as pl
from jax.experimental.pallas import tpu as pltpu

BLOCK_C = 16384


def _body(x_ref, w_ref, b_ref, o_ref):
    acc = jax.lax.dot_general(
        x_ref[...], w_ref[...],
        dimension_numbers=(((1,), (1,)), ((), ())),
        preferred_element_type=jnp.float32,
    )
    o_ref[...] = acc + b_ref[...]


def kernel(x, weight, bias):
    n_classes = weight.shape[0]
    grid = (pl.cdiv(n_classes, BLOCK_C),)
    bias2d = bias.reshape(1, n_classes)
    out = pl.pallas_call(
        _body,
        grid=grid,
        in_specs=[
            pl.BlockSpec((x.shape[0], x.shape[1]), lambda i: (0, 0)),
            pl.BlockSpec((BLOCK_C, weight.shape[1]), lambda i: (i, 0)),
            pl.BlockSpec((1, BLOCK_C), lambda i: (0, i)),
        ],
        out_specs=pl.BlockSpec((x.shape[0], BLOCK_C), lambda i: (0, i)),
        out_shape=jax.ShapeDtypeStruct((x.shape[0], n_classes), jnp.float32),
        compiler_params=pltpu.CompilerParams(
            dimension_semantics=("parallel",),
        ),
    )(x, weight, bias2d)
    return out
